# Initial kernel scaffold; baseline (speedup 1.0000x reference)
import functools

import jax
import jax.numpy as jnp
from jax import lax
from jax.experimental import pallas as pl
from jax.experimental.pallas import tpu as pltpu
from jax.experimental.pallas import tpu_sc as plsc

_NUM_SEQS = 26
_VOCAB = 100000
_DIM = 32
_BATCH = 1024
_HIST = 20

_NC, _NS = 2, 16
_NW = _NC * _NS
_ROWS = _BATCH * _HIST * _NUM_SEQS
_R_W = _ROWS // _NW
_CH = 416
_NCH = _R_W // _CH
_SLOTS = 8
_LOOK = 4


def _make_gather():
    mesh = plsc.VectorSubcoreMesh(core_axis_name="c", subcore_axis_name="s")

    @functools.partial(
        pl.kernel,
        mesh=mesh,
        out_type=jax.ShapeDtypeStruct((_ROWS, _DIM), jnp.float32),
        scratch_types=(
            [pltpu.VMEM((_R_W,), jnp.int32)]
            + [pltpu.VMEM((_CH, _DIM), jnp.float32) for _ in range(_SLOTS)]
            + [pltpu.SemaphoreType.DMA for _ in range(_SLOTS)]
        ),
    )
    def gather_kernel(idx_hbm, tab_hbm, out_hbm, idx_v, *scratch):
        bufs = scratch[:_SLOTS]
        sems = scratch[_SLOTS:]
        wid = lax.axis_index("s") * _NC + lax.axis_index("c")
        base = wid * _R_W

        pltpu.sync_copy(idx_hbm.at[pl.ds(base, _R_W)], idx_v)

        def g_start(c, slot):
            pltpu.make_async_copy(
                tab_hbm.at[idx_v.at[pl.ds(c * _CH, _CH)]], bufs[slot],
                sems[slot],
            ).start()

        def g_wait(c, slot):
            pltpu.make_async_copy(
                tab_hbm.at[idx_v.at[pl.ds(c * _CH, _CH)]], bufs[slot],
                sems[slot],
            ).wait()

        def put(c, slot):
            pltpu.sync_copy(bufs[slot], out_hbm.at[pl.ds(base + c * _CH, _CH)])

        for c in range(_LOOK):
            g_start(c, c % _SLOTS)

        def block(b, carry):
            for k in range(_SLOTS):
                c = b * _SLOTS + k
                g_wait(c, k)
                put(c, k)
                nc = c + _LOOK

                @pl.when(nc < _NCH)
                def _():
                    g_start(nc, (k + _LOOK) % _SLOTS)
            return carry

        lax.fori_loop(0, _NCH // _SLOTS, block, 0)

    return gather_kernel


_gather = _make_gather()


def kernel(users_seqs, tables):
    offs = (jnp.arange(_NUM_SEQS, dtype=jnp.int32) * _VOCAB)[None, None, :]
    idx = (jnp.transpose(users_seqs, (0, 2, 1)) + offs).reshape(_ROWS)
    tab = tables.reshape(_NUM_SEQS * _VOCAB, _DIM)
    out = _gather(idx, tab)
    return out.reshape(_BATCH, _HIST, _NUM_SEQS * _DIM)


# speedup vs baseline: 1.3128x; 1.3128x over previous
import functools

import jax
import jax.numpy as jnp
from jax import lax
from jax.experimental import pallas as pl
from jax.experimental.pallas import tpu as pltpu
from jax.experimental.pallas import tpu_sc as plsc

_NUM_SEQS = 26
_VOCAB = 100000
_DIM = 32
_BATCH = 1024
_HIST = 20

_NC, _NS = 2, 16
_NW = _NC * _NS
_ROWS = _BATCH * _HIST * _NUM_SEQS
_R_W = _ROWS // _NW
_CH = 416
_NCH = _R_W // _CH
_SLOTS = 8
_LOOK = 4


def _make_gather():
    mesh = plsc.VectorSubcoreMesh(core_axis_name="c", subcore_axis_name="s")

    @functools.partial(
        pl.kernel,
        mesh=mesh,
        out_type=jax.ShapeDtypeStruct((_ROWS, _DIM), jnp.float32),
        scratch_types=(
            [pltpu.VMEM((_R_W,), jnp.int32)]
            + [pltpu.VMEM((_CH, _DIM), jnp.float32) for _ in range(_SLOTS)]
            + [pltpu.SemaphoreType.DMA for _ in range(_SLOTS)]
        ),
        compiler_params=pltpu.CompilerParams(use_tc_tiling_on_sc=False),
    )
    def gather_kernel(idx_hbm, tab_hbm, out_hbm, idx_v, *scratch):
        bufs = scratch[:_SLOTS]
        sems = scratch[_SLOTS:]
        wid = lax.axis_index("s") * _NC + lax.axis_index("c")
        base = wid * _R_W

        pltpu.sync_copy(idx_hbm.at[pl.ds(base, _R_W)], idx_v)

        def g_start(c, slot):
            pltpu.make_async_copy(
                tab_hbm.at[idx_v.at[pl.ds(c * _CH, _CH)]], bufs[slot],
                sems[slot],
            ).start()

        def g_wait(c, slot):
            pltpu.make_async_copy(
                tab_hbm.at[idx_v.at[pl.ds(c * _CH, _CH)]], bufs[slot],
                sems[slot],
            ).wait()

        def put(c, slot):
            pltpu.sync_copy(bufs[slot], out_hbm.at[pl.ds(base + c * _CH, _CH)])

        for c in range(_LOOK):
            g_start(c, c % _SLOTS)

        def block(b, carry):
            for k in range(_SLOTS):
                c = b * _SLOTS + k
                g_wait(c, k)
                put(c, k)
                nc = c + _LOOK

                @pl.when(nc < _NCH)
                def _():
                    g_start(nc, (k + _LOOK) % _SLOTS)
            return carry

        lax.fori_loop(0, _NCH // _SLOTS, block, 0)

    return gather_kernel


_gather = _make_gather()


def kernel(users_seqs, tables):
    offs = (jnp.arange(_NUM_SEQS, dtype=jnp.int32) * _VOCAB)[None, None, :]
    idx = (jnp.transpose(users_seqs, (0, 2, 1)) + offs).reshape(_ROWS)
    tab = tables.reshape(_NUM_SEQS * _VOCAB, _DIM)
    out = _gather(idx, tab)
    return out.reshape(_BATCH, _HIST, _NUM_SEQS * _DIM)
